# Initial kernel scaffold; baseline (speedup 1.0000x reference)
import jax
import jax.numpy as jnp
from jax.experimental import pallas as pl


def kernel(input_points, planes):
    raise NotImplementedError("write your pallas kernel here")


# speedup vs baseline: 1.8660x; 1.8660x over previous
import jax
import jax.numpy as jnp
from jax.experimental import pallas as pl

_BN = 512


def _lsh_block_kernel(x_ref, pt_ref, o_ref):
    x = x_ref[...]
    for t in range(o_ref.shape[0]):
        acc = jax.lax.dot_general(
            x, pt_ref[t],
            dimension_numbers=(((1,), (0,)), ((), ())),
            preferred_element_type=jnp.float32,
        )
        o_ref[t] = jnp.where(acc < 0, jnp.float32(0.0), jnp.float32(1.0))


def kernel(input_points, planes):
    n, d = input_points.shape
    t, h, _ = planes.shape
    pt = jnp.swapaxes(planes, 1, 2)
    return pl.pallas_call(
        _lsh_block_kernel,
        grid=(n // _BN,),
        in_specs=[
            pl.BlockSpec((_BN, d), lambda i: (i, 0)),
            pl.BlockSpec((t, d, h), lambda i: (0, 0, 0)),
        ],
        out_specs=pl.BlockSpec((t, _BN, h), lambda i: (0, i, 0)),
        out_shape=jax.ShapeDtypeStruct((t, n, h), jnp.float32),
    )(input_points, pt)


# speedup vs baseline: 2.2881x; 1.2262x over previous
import jax
import jax.numpy as jnp
from jax.experimental import pallas as pl

_BN = 1024


def _lsh_block_kernel(x_ref, pt_ref, o_ref):
    x = x_ref[...]
    for t in range(o_ref.shape[0]):
        acc = jax.lax.dot_general(
            x, pt_ref[t],
            dimension_numbers=(((1,), (0,)), ((), ())),
            preferred_element_type=jnp.float32,
        )
        o_ref[t] = jnp.where(acc < 0, jnp.float32(0.0), jnp.float32(1.0))


def kernel(input_points, planes):
    n, d = input_points.shape
    t, h, _ = planes.shape
    pt = jnp.swapaxes(planes, 1, 2)
    return pl.pallas_call(
        _lsh_block_kernel,
        grid=(n // _BN,),
        in_specs=[
            pl.BlockSpec((_BN, d), lambda i: (i, 0)),
            pl.BlockSpec((t, d, h), lambda i: (0, 0, 0)),
        ],
        out_specs=pl.BlockSpec((t, _BN, h), lambda i: (0, i, 0)),
        out_shape=jax.ShapeDtypeStruct((t, n, h), jnp.float32),
    )(input_points, pt)


# speedup vs baseline: 2.4861x; 1.0865x over previous
import jax
import jax.numpy as jnp
from jax.experimental import pallas as pl

_BN = 2048


def _lsh_block_kernel(x_ref, pt_ref, o_ref):
    x = x_ref[...]
    for t in range(o_ref.shape[0]):
        acc = jax.lax.dot_general(
            x, pt_ref[t],
            dimension_numbers=(((1,), (0,)), ((), ())),
            preferred_element_type=jnp.float32,
        )
        o_ref[t] = jnp.where(acc < 0, jnp.float32(0.0), jnp.float32(1.0))


def kernel(input_points, planes):
    n, d = input_points.shape
    t, h, _ = planes.shape
    pt = jnp.swapaxes(planes, 1, 2)
    return pl.pallas_call(
        _lsh_block_kernel,
        grid=(n // _BN,),
        in_specs=[
            pl.BlockSpec((_BN, d), lambda i: (i, 0)),
            pl.BlockSpec((t, d, h), lambda i: (0, 0, 0)),
        ],
        out_specs=pl.BlockSpec((t, _BN, h), lambda i: (0, i, 0)),
        out_shape=jax.ShapeDtypeStruct((t, n, h), jnp.float32),
    )(input_points, pt)


# speedup vs baseline: 2.5223x; 1.0146x over previous
import jax
import jax.numpy as jnp
from jax.experimental import pallas as pl

_BN = 4096


def _lsh_block_kernel(x_ref, pt_ref, o_ref):
    x = x_ref[...]
    for t in range(o_ref.shape[0]):
        acc = jax.lax.dot_general(
            x, pt_ref[t],
            dimension_numbers=(((1,), (0,)), ((), ())),
            preferred_element_type=jnp.float32,
        )
        o_ref[t] = jnp.where(acc < 0, jnp.float32(0.0), jnp.float32(1.0))


def kernel(input_points, planes):
    n, d = input_points.shape
    t, h, _ = planes.shape
    pt = jnp.swapaxes(planes, 1, 2)
    return pl.pallas_call(
        _lsh_block_kernel,
        grid=(n // _BN,),
        in_specs=[
            pl.BlockSpec((_BN, d), lambda i: (i, 0)),
            pl.BlockSpec((t, d, h), lambda i: (0, 0, 0)),
        ],
        out_specs=pl.BlockSpec((t, _BN, h), lambda i: (0, i, 0)),
        out_shape=jax.ShapeDtypeStruct((t, n, h), jnp.float32),
    )(input_points, pt)


# speedup vs baseline: 2.7425x; 1.0873x over previous
import jax
import jax.numpy as jnp
from jax.experimental import pallas as pl

_BN = 4096


def _lsh_block_kernel(x_ref, p_ref, o_ref):
    x = x_ref[...]
    for t in range(o_ref.shape[0]):
        acc = jax.lax.dot_general(
            x, p_ref[t],
            dimension_numbers=(((1,), (1,)), ((), ())),
            preferred_element_type=jnp.float32,
        )
        o_ref[t] = jnp.where(acc < 0, jnp.float32(0.0), jnp.float32(1.0))


def kernel(input_points, planes):
    n, d = input_points.shape
    t, h, _ = planes.shape
    return pl.pallas_call(
        _lsh_block_kernel,
        grid=(n // _BN,),
        in_specs=[
            pl.BlockSpec((_BN, d), lambda i: (i, 0)),
            pl.BlockSpec((t, h, d), lambda i: (0, 0, 0)),
        ],
        out_specs=pl.BlockSpec((t, _BN, h), lambda i: (0, i, 0)),
        out_shape=jax.ShapeDtypeStruct((t, n, h), jnp.float32),
    )(input_points, planes)
